# Initial kernel scaffold; baseline (speedup 1.0000x reference)
import jax
import jax.numpy as jnp
from jax.experimental import pallas as pl


def kernel(in_feat, edge_index, W1, b1, W2, b2, W3, b3):
    raise NotImplementedError("write your pallas kernel here")


# speedup vs baseline: 4.9270x; 4.9270x over previous
import functools

import jax
import jax.numpy as jnp
from jax import lax
from jax.experimental import pallas as pl
from jax.experimental.pallas import tpu as pltpu
from jax.experimental.pallas import tpu_sc as plsc

NUM_CORES = 2
NUM_SUBCORES = 16
NW = NUM_CORES * NUM_SUBCORES
CHUNK = 128


@functools.partial(jax.jit, static_argnums=(3, 4))
def _segsum(h, src, dst, n_pad, cpt):
  d = h.shape[1]
  rows_per_tile = n_pad // NUM_SUBCORES
  zcopies = rows_per_tile // CHUNK

  mesh = plsc.VectorSubcoreMesh(
      core_axis_name="c", subcore_axis_name="s",
      num_cores=NUM_CORES, num_subcores=NUM_SUBCORES)

  @functools.partial(
      pl.kernel,
      out_type=jax.ShapeDtypeStruct((NUM_CORES, n_pad, d), jnp.float32),
      mesh=mesh,
      scratch_types=[
          pltpu.VMEM((cpt, CHUNK), jnp.int32),
          pltpu.VMEM((cpt, CHUNK), jnp.int32),
          pltpu.VMEM((CHUNK, d), jnp.float32),
          pltpu.VMEM_SHARED((n_pad, d), jnp.float32),
          pltpu.SemaphoreType.DMA,
      ],
  )
  def seg(h_hbm, src_hbm, dst_hbm, out_hbm, src_v, dst_v, msgs, agg, sem):
    cid = lax.axis_index("c")
    sid = lax.axis_index("s")
    wid = cid * NUM_SUBCORES + sid

    pltpu.sync_copy(src_hbm.at[wid], src_v)
    pltpu.sync_copy(dst_hbm.at[wid], dst_v)

    zero = jnp.zeros((16,), jnp.float32)

    def zbody(r, carry):
      for k in range(d // 16):
        msgs[r, pl.ds(k * 16, 16)] = zero
      return carry

    lax.fori_loop(0, CHUNK, zbody, 0)
    base = sid * rows_per_tile
    for i in range(zcopies):
      pltpu.sync_copy(msgs, agg.at[pl.ds(base + i * CHUNK, CHUNK)])
    plsc.subcore_barrier()

    def body(j, carry):
      pltpu.async_copy(h_hbm.at[src_v.at[j]], msgs, sem).wait()
      pltpu.sync_copy(msgs, agg.at[dst_v.at[j]], add=True)
      return carry

    lax.fori_loop(0, cpt, body, 0)
    plsc.subcore_barrier()

    for i in range(zcopies):
      sl = pl.ds(base + i * CHUNK, CHUNK)
      pltpu.sync_copy(agg.at[sl], out_hbm.at[cid, sl])

  return seg(h, src, dst)


@functools.partial(jax.jit, static_argnums=(3,))
def _linear(p, w, b, leaky):
  n_pad, d = p.shape[1], p.shape[2]
  blk = 512
  grid = n_pad // blk
  b2 = b.reshape(1, d)

  def body(p_ref, w_ref, b_ref, o_ref):
    s = p_ref[0] + p_ref[1]
    y = jnp.dot(s, w_ref[...], preferred_element_type=jnp.float32)
    y = y + b_ref[...]
    if leaky:
      y = jnp.where(y >= 0, y, 0.1 * y)
    o_ref[...] = y

  return pl.pallas_call(
      body,
      grid=(grid,),
      in_specs=[
          pl.BlockSpec((NUM_CORES, blk, d), lambda i: (0, i, 0)),
          pl.BlockSpec((d, d), lambda i: (0, 0)),
          pl.BlockSpec((1, d), lambda i: (0, 0)),
      ],
      out_specs=pl.BlockSpec((blk, d), lambda i: (i, 0)),
      out_shape=jax.ShapeDtypeStruct((n_pad, d), jnp.float32),
  )(p, w, b2)


def kernel(in_feat, edge_index, W1, b1, W2, b2, W3, b3):
  n, d = in_feat.shape
  e = edge_index.shape[1]

  cpt = -(-e // (NW * CHUNK))
  e_pad = NW * CHUNK * cpt
  unit = NUM_SUBCORES * CHUNK
  n_pad = (n // unit + 1) * unit

  src = jnp.concatenate(
      [edge_index[0], jnp.zeros((e_pad - e,), jnp.int32)]).reshape(
          NW, cpt, CHUNK)
  dst = jnp.concatenate(
      [edge_index[1], jnp.full((e_pad - e,), n, jnp.int32)]).reshape(
          NW, cpt, CHUNK)
  h = jnp.pad(in_feat, ((0, n_pad - n), (0, 0)))

  p = _segsum(h, src, dst, n_pad, cpt)
  h = _linear(p, W1, b1, True)
  p = _segsum(h, src, dst, n_pad, cpt)
  h = _linear(p, W2, b2, True)
  p = _segsum(h, src, dst, n_pad, cpt)
  h = _linear(p, W3, b3, False)
  return h[:n]
